# baseline (speedup 1.0000x reference)
import functools

import jax
import jax.numpy as jnp
from jax.experimental import pallas as pl

K = 64
_GB = 256
_NB = 2048


def _dist_body(qb, kb, qn, kn, mk, out):
    s = jnp.dot(qb[0], kb[0], preferred_element_type=jnp.float32)
    d2 = (qn[0] + kn[0]) + s
    dist = jnp.sqrt(jnp.maximum(d2, 0.0))
    out[0] = jnp.where(mk[0] == 0, jnp.inf, dist)


def _masked_dists(qb, kb, qn, kn, mk):
    B, G, _ = qb.shape
    N = kb.shape[2]
    grid = (B, G // _GB, N // _NB)
    return pl.pallas_call(
        _dist_body,
        grid=grid,
        in_specs=[
            pl.BlockSpec((1, _GB, 8), lambda b, g, n: (b, g, 0)),
            pl.BlockSpec((1, 8, _NB), lambda b, g, n: (b, 0, n)),
            pl.BlockSpec((1, _GB, 1), lambda b, g, n: (b, g, 0)),
            pl.BlockSpec((1, 1, _NB), lambda b, g, n: (b, 0, n)),
            pl.BlockSpec((1, 1, _NB), lambda b, g, n: (b, 0, n)),
        ],
        out_specs=pl.BlockSpec((1, _GB, _NB), lambda b, g, n: (b, g, n)),
        out_shape=jax.ShapeDtypeStruct((B, G, N), jnp.float32),
    )(qb, kb, qn, kn, mk)


@jax.jit
def kernel(grasp_translations, scene_xyz, scene_mask):
    q = grasp_translations
    kp = scene_xyz
    B, G, _ = q.shape
    N = kp.shape[1]
    qn = (q * q).sum(-1, keepdims=True)
    kn = (kp * kp).sum(-1)[:, None, :]
    qb = jnp.pad(-2.0 * q, ((0, 0), (0, 0), (0, 5))).astype(jnp.bfloat16)
    kb = jnp.swapaxes(jnp.pad(kp, ((0, 0), (0, 0), (0, 5))), 1, 2).astype(jnp.bfloat16)
    mk = scene_mask[:, None, :]
    dists = _masked_dists(qb, kb, qn, kn, mk)
    neg_vals, indices = jax.lax.top_k(-dists, K)
    distances = -neg_vals
    local_mask = jnp.isfinite(distances).astype(jnp.float32)
    return (indices, local_mask)


# speedup vs baseline: 38.2897x; 38.2897x over previous
import functools

import jax
import jax.numpy as jnp
from jax import lax
from jax.experimental import pallas as pl
from jax.experimental.pallas import tpu as pltpu
from jax.experimental.pallas import tpu_sc as plsc

K = 64
NCH = 512
_GB = 256
_NB = 2048
_QPW = 128
_CCAP = 2048 + 80


def _dist_body(qb, kb, qn, kn, mk, dout, mout):
    n = pl.program_id(2)
    s = jnp.dot(qb[0], kb[0], preferred_element_type=jnp.float32)
    d2 = (qn[0] + kn[0]) + s
    dist = jnp.sqrt(jnp.maximum(d2, 0.0))
    dist = jnp.where(mk[0] == 0, jnp.inf, dist)
    dout[0] = dist
    h = jnp.minimum(dist[:, :1024], dist[:, 1024:])
    h = jnp.minimum(h[:, :512], h[:, 512:])

    @pl.when(n == 0)
    def _():
        mout[0] = h

    @pl.when(n != 0)
    def _():
        mout[0] = jnp.minimum(mout[0], h)


def _masked_dists(qb, kb, qn, kn, mk):
    B, G, _ = qb.shape
    N = kb.shape[2]
    grid = (B, G // _GB, N // _NB)
    return pl.pallas_call(
        _dist_body,
        grid=grid,
        in_specs=[
            pl.BlockSpec((1, _GB, 8), lambda b, g, n: (b, g, 0)),
            pl.BlockSpec((1, 8, _NB), lambda b, g, n: (b, 0, n)),
            pl.BlockSpec((1, _GB, 1), lambda b, g, n: (b, g, 0)),
            pl.BlockSpec((1, 1, _NB), lambda b, g, n: (b, 0, n)),
            pl.BlockSpec((1, 1, _NB), lambda b, g, n: (b, 0, n)),
        ],
        out_specs=[
            pl.BlockSpec((1, _GB, _NB), lambda b, g, n: (b, g, n)),
            pl.BlockSpec((1, _GB, NCH), lambda b, g, n: (b, g, 0)),
        ],
        out_shape=[
            jax.ShapeDtypeStruct((B, G, N), jnp.float32),
            jax.ShapeDtypeStruct((B, G, NCH), jnp.float32),
        ],
    )(qb, kb, qn, kn, mk)


def _rev(x):
    return lax.rev(x, (0,))


def _ce(ka, va, kb, vb):
    c = ka <= kb
    return (jnp.where(c, ka, kb), jnp.where(c, va, vb),
            jnp.where(c, kb, ka), jnp.where(c, vb, va))


def _vsort(k, v):
    return plsc.sort_key_val(k, v)


def _sort64(ks, vs):
    ks = [None] * 4 if ks is None else list(ks)
    vs = list(vs)
    for i in range(4):
        ks[i], vs[i] = _vsort(ks[i], vs[i])
    runs = []
    for a, b in ((0, 1), (2, 3)):
        lo_k, lo_v, hi_k, hi_v = _ce(ks[a], vs[a], _rev(ks[b]), _rev(vs[b]))
        lo_k, lo_v = _vsort(lo_k, lo_v)
        hi_k, hi_v = _vsort(hi_k, hi_v)
        runs.append((lo_k, lo_v, hi_k, hi_v))
    (a0k, a0v, a1k, a1v), (b0k, b0v, b1k, b1v) = runs
    l0k, l0v, h0k, h0v = _ce(a0k, a0v, _rev(b1k), _rev(b1v))
    l1k, l1v, h1k, h1v = _ce(a1k, a1v, _rev(b0k), _rev(b0v))
    m0k, m0v, m1k, m1v = _ce(l0k, l0v, l1k, l1v)
    m0k, m0v = _vsort(m0k, m0v)
    m1k, m1v = _vsort(m1k, m1v)
    n0k, n0v, n1k, n1v = _ce(h0k, h0v, h1k, h1v)
    n0k, n0v = _vsort(n0k, n0v)
    n1k, n1v = _vsort(n1k, n1v)
    return [m0k, m1k, n0k, n1k], [m0v, m1v, n0v, n1v]


def _merge_keep_low(rk, rv, ak, av):
    lk, lv = [], []
    for i in range(4):
        k_lo, v_lo, _, _ = _ce(rk[i], rv[i], _rev(ak[3 - i]), _rev(av[3 - i]))
        lk.append(k_lo)
        lv.append(v_lo)
    x0k, x0v, x2k, x2v = _ce(lk[0], lv[0], lk[2], lv[2])
    x1k, x1v, x3k, x3v = _ce(lk[1], lv[1], lk[3], lv[3])
    y0k, y0v, y1k, y1v = _ce(x0k, x0v, x1k, x1v)
    y2k, y2v, y3k, y3v = _ce(x2k, x2v, x3k, x3v)
    ok, ov = [], []
    for kk, vv in ((y0k, y0v), (y1k, y1v), (y2k, y2v), (y3k, y3v)):
        kk, vv = _vsort(kk, vv)
        ok.append(kk)
        ov.append(vv)
    return ok, ov


def _gat(x, idx):
    return x.at[idx].get(mode="promise_in_bounds")


def _stable_fix(rk, rv, iota):
    i32 = jnp.int32
    pu = jnp.minimum(iota + 1, 15)
    pd = jnp.maximum(iota - 1, 0)
    last = iota == 15
    first = iota == 0
    evenm = (iota % 2) == 0
    zero16 = jnp.zeros((16,), i32)
    fifteen = jnp.full((16,), 15, i32)
    for phase in range(4):
        par = evenm if phase % 2 == 0 else jnp.logical_not(evenm)
        old = list(rv)
        sws, vns = [], []
        for j in range(4):
            ks = _gat(rk[j], pu)
            vs = _gat(old[j], pu)
            if j < 3:
                kn = jnp.where(last, _gat(rk[j + 1], zero16), ks)
                vn = jnp.where(last, _gat(old[j + 1], zero16), vs)
            else:
                kn = jnp.where(last, jnp.full((16,), jnp.inf, jnp.float32), ks)
                vn = jnp.where(last, zero16, vs)
            sw = (rk[j] == kn) & (old[j] > vn) & par
            sws.append(sw)
            vns.append(vn)
        for j in range(4):
            swi = sws[j].astype(i32)
            sp = _gat(swi, pd)
            vsp = _gat(old[j], pd)
            if j > 0:
                sp = jnp.where(first, _gat(sws[j - 1].astype(i32), fifteen), sp)
                vsp = jnp.where(first, _gat(old[j - 1], fifteen), vsp)
            else:
                sp = jnp.where(first, zero16, sp)
            rv[j] = jnp.where(sws[j], vns[j],
                              jnp.where(sp > 0, vsp, old[j]))
    return rv


def _flat(rk, rv):
    return tuple(rk) + tuple(rv)


def _unflat(c):
    return list(c[:4]), list(c[4:])


def _sc_body(dist_hbm, m_hbm, oidx_hbm, omsk_hbm,
             mrow, drow, candv, candi, obuf_i, obuf_m, sem_d, sem_m):
    wid = lax.axis_index("s") * 2 + lax.axis_index("c")
    iota = lax.iota(jnp.int32, 16)
    inf16 = jnp.full((16,), jnp.inf, jnp.float32)
    zero16 = jnp.zeros((16,), jnp.int32)

    def qbody(i, _carry):
        q = wid * _QPW + i
        cpd = pltpu.async_copy(dist_hbm.at[q], drow, sem_d)
        cpm = pltpu.async_copy(m_hbm.at[q], mrow, sem_m)
        cpm.wait()

        rk, rv = [inf16] * 4, [zero16] * 4
        for blk in range(NCH // 64):
            ks = [mrow[pl.ds(blk * 64 + j * 16, 16)] for j in range(4)]
            vs = [iota + (blk * 64 + j * 16) for j in range(4)]
            ak, av = _sort64(ks, vs)
            rk, rv = _merge_keep_low(rk, rv, ak, av)
        t = jnp.max(rk[3])

        cpd.wait()

        def cbody(j2, off):
            o = off
            for j in range(4):
                idxv = rv[j] + j2 * NCH
                vals = plsc.load_gather(drow, [idxv])
                msk = vals <= t
                cnt = jnp.max(plsc.all_reduce_population_count(msk))
                keyv = jnp.where(vals == 0.0,
                                 idxv.astype(jnp.float32) * jnp.float32(1e-30),
                                 vals)
                plsc.store_compressed(candv.at[pl.ds(o, 16)], keyv, mask=msk)
                plsc.store_compressed(candi.at[pl.ds(o, 16)], idxv, mask=msk)
                o = o + cnt
            return o

        off = lax.fori_loop(0, 32, cbody, jnp.int32(0))

        for j in range(4):
            candv[pl.ds(off + j * 16, 16)] = inf16
            candi[pl.ds(off + j * 16, 16)] = zero16

        nb = (off + 63) // 64

        def fbody(b, carry):
            rk2, rv2 = _unflat(carry)
            ks = [candv[pl.ds(b * 64 + j * 16, 16)] for j in range(4)]
            vs = [candi[pl.ds(b * 64 + j * 16, 16)] for j in range(4)]
            ak, av = _sort64(ks, vs)
            rk2, rv2 = _merge_keep_low(rk2, rv2, ak, av)
            return _flat(rk2, rv2)

        fin = lax.fori_loop(0, nb, fbody, _flat([inf16] * 4, [zero16] * 4))
        rk2, rv2 = _unflat(fin)
        rv2 = _stable_fix(rk2, rv2, iota)

        for j in range(4):
            obuf_i[pl.ds(j * 16, 16)] = rv2[j]
            obuf_m[pl.ds(j * 16, 16)] = jnp.where(
                rk2[j] < jnp.inf, jnp.float32(1.0), jnp.float32(0.0))
        pltpu.sync_copy(obuf_i, oidx_hbm.at[q])
        pltpu.sync_copy(obuf_m, omsk_hbm.at[q])
        return _carry

    lax.fori_loop(0, _QPW, qbody, jnp.int32(0))


def _sc_select(dist, mins):
    BG = dist.shape[0]
    mesh = plsc.VectorSubcoreMesh(core_axis_name="c", subcore_axis_name="s")
    f = pl.kernel(
        _sc_body,
        mesh=mesh,
        compiler_params=pltpu.CompilerParams(needs_layout_passes=False),
        out_type=[
            jax.ShapeDtypeStruct((BG, K), jnp.int32),
            jax.ShapeDtypeStruct((BG, K), jnp.float32),
        ],
        scratch_types=[
            pltpu.VMEM((NCH,), jnp.float32),
            pltpu.VMEM((16384,), jnp.float32),
            pltpu.VMEM((_CCAP,), jnp.float32),
            pltpu.VMEM((_CCAP,), jnp.int32),
            pltpu.VMEM((K,), jnp.int32),
            pltpu.VMEM((K,), jnp.float32),
            pltpu.SemaphoreType.DMA,
            pltpu.SemaphoreType.DMA,
        ],
    )
    return f(dist, mins)


@jax.jit
def kernel(grasp_translations, scene_xyz, scene_mask):
    q = grasp_translations
    kp = scene_xyz
    B, G, _ = q.shape
    N = kp.shape[1]
    qn = (q * q).sum(-1, keepdims=True)
    kn = (kp * kp).sum(-1)[:, None, :]
    qb = jnp.pad(-2.0 * q, ((0, 0), (0, 0), (0, 5))).astype(jnp.bfloat16)
    kb = jnp.swapaxes(jnp.pad(kp, ((0, 0), (0, 0), (0, 5))), 1, 2).astype(jnp.bfloat16)
    mk = scene_mask[:, None, :]
    dists, mins = _masked_dists(qb, kb, qn, kn, mk)
    idx, msk = _sc_select(dists.reshape(B * G, N), mins.reshape(B * G, NCH))
    return (idx.reshape(B, G, K), msk.reshape(B, G, K))


# speedup vs baseline: 39.1995x; 1.0238x over previous
import functools

import jax
import jax.numpy as jnp
from jax import lax
from jax.experimental import pallas as pl
from jax.experimental.pallas import tpu as pltpu
from jax.experimental.pallas import tpu_sc as plsc

K = 64
NCH = 512
_GB = 256
_NB = 2048
_QPW = 128
_CCAP = 2048 + 80


def _dist_body(qb, kb, qn, kn, mk, dout, mout):
    n = pl.program_id(2)
    s = jnp.dot(qb[0], kb[0], preferred_element_type=jnp.float32)
    d2 = (qn[0] + kn[0]) + s
    dist = jnp.sqrt(jnp.maximum(d2, 0.0))
    dist = jnp.where(mk[0] == 0, jnp.inf, dist)
    dout[0] = dist
    h = jnp.minimum(dist[:, :1024], dist[:, 1024:])
    h = jnp.minimum(h[:, :512], h[:, 512:])

    @pl.when(n == 0)
    def _():
        mout[0] = h

    @pl.when(n != 0)
    def _():
        mout[0] = jnp.minimum(mout[0], h)


def _masked_dists(qb, kb, qn, kn, mk):
    B, G, _ = qb.shape
    N = kb.shape[2]
    grid = (B, G // _GB, N // _NB)
    return pl.pallas_call(
        _dist_body,
        grid=grid,
        in_specs=[
            pl.BlockSpec((1, _GB, 8), lambda b, g, n: (b, g, 0)),
            pl.BlockSpec((1, 8, _NB), lambda b, g, n: (b, 0, n)),
            pl.BlockSpec((1, _GB, 1), lambda b, g, n: (b, g, 0)),
            pl.BlockSpec((1, 1, _NB), lambda b, g, n: (b, 0, n)),
            pl.BlockSpec((1, 1, _NB), lambda b, g, n: (b, 0, n)),
        ],
        out_specs=[
            pl.BlockSpec((1, _GB, _NB), lambda b, g, n: (b, g, n)),
            pl.BlockSpec((1, _GB, NCH), lambda b, g, n: (b, g, 0)),
        ],
        out_shape=[
            jax.ShapeDtypeStruct((B, G, N), jnp.float32),
            jax.ShapeDtypeStruct((B, G, NCH), jnp.float32),
        ],
    )(qb, kb, qn, kn, mk)


def _rev(x):
    return lax.rev(x, (0,))


def _ce(ka, va, kb, vb):
    c = ka <= kb
    return (jnp.where(c, ka, kb), jnp.where(c, va, vb),
            jnp.where(c, kb, ka), jnp.where(c, vb, va))


def _vsort(k, v):
    return plsc.sort_key_val(k, v)


def _sort64(ks, vs):
    ks = [None] * 4 if ks is None else list(ks)
    vs = list(vs)
    for i in range(4):
        ks[i], vs[i] = _vsort(ks[i], vs[i])
    runs = []
    for a, b in ((0, 1), (2, 3)):
        lo_k, lo_v, hi_k, hi_v = _ce(ks[a], vs[a], _rev(ks[b]), _rev(vs[b]))
        lo_k, lo_v = _vsort(lo_k, lo_v)
        hi_k, hi_v = _vsort(hi_k, hi_v)
        runs.append((lo_k, lo_v, hi_k, hi_v))
    (a0k, a0v, a1k, a1v), (b0k, b0v, b1k, b1v) = runs
    l0k, l0v, h0k, h0v = _ce(a0k, a0v, _rev(b1k), _rev(b1v))
    l1k, l1v, h1k, h1v = _ce(a1k, a1v, _rev(b0k), _rev(b0v))
    m0k, m0v, m1k, m1v = _ce(l0k, l0v, l1k, l1v)
    m0k, m0v = _vsort(m0k, m0v)
    m1k, m1v = _vsort(m1k, m1v)
    n0k, n0v, n1k, n1v = _ce(h0k, h0v, h1k, h1v)
    n0k, n0v = _vsort(n0k, n0v)
    n1k, n1v = _vsort(n1k, n1v)
    return [m0k, m1k, n0k, n1k], [m0v, m1v, n0v, n1v]


def _merge_keep_low(rk, rv, ak, av):
    lk, lv = [], []
    for i in range(4):
        k_lo, v_lo, _, _ = _ce(rk[i], rv[i], _rev(ak[3 - i]), _rev(av[3 - i]))
        lk.append(k_lo)
        lv.append(v_lo)
    x0k, x0v, x2k, x2v = _ce(lk[0], lv[0], lk[2], lv[2])
    x1k, x1v, x3k, x3v = _ce(lk[1], lv[1], lk[3], lv[3])
    y0k, y0v, y1k, y1v = _ce(x0k, x0v, x1k, x1v)
    y2k, y2v, y3k, y3v = _ce(x2k, x2v, x3k, x3v)
    ok, ov = [], []
    for kk, vv in ((y0k, y0v), (y1k, y1v), (y2k, y2v), (y3k, y3v)):
        kk, vv = _vsort(kk, vv)
        ok.append(kk)
        ov.append(vv)
    return ok, ov


def _gat(x, idx):
    return x.at[idx].get(mode="promise_in_bounds")


def _stable_fix(rk, rv, iota):
    i32 = jnp.int32
    pu = jnp.minimum(iota + 1, 15)
    pd = jnp.maximum(iota - 1, 0)
    last = iota == 15
    first = iota == 0
    evenm = (iota % 2) == 0
    zero16 = jnp.zeros((16,), i32)
    fifteen = jnp.full((16,), 15, i32)
    for phase in range(4):
        par = evenm if phase % 2 == 0 else jnp.logical_not(evenm)
        old = list(rv)
        sws, vns = [], []
        for j in range(4):
            ks = _gat(rk[j], pu)
            vs = _gat(old[j], pu)
            if j < 3:
                kn = jnp.where(last, _gat(rk[j + 1], zero16), ks)
                vn = jnp.where(last, _gat(old[j + 1], zero16), vs)
            else:
                kn = jnp.where(last, jnp.full((16,), jnp.inf, jnp.float32), ks)
                vn = jnp.where(last, zero16, vs)
            sw = (rk[j] == kn) & (old[j] > vn) & par
            sws.append(sw)
            vns.append(vn)
        for j in range(4):
            swi = sws[j].astype(i32)
            sp = _gat(swi, pd)
            vsp = _gat(old[j], pd)
            if j > 0:
                sp = jnp.where(first, _gat(sws[j - 1].astype(i32), fifteen), sp)
                vsp = jnp.where(first, _gat(old[j - 1], fifteen), vsp)
            else:
                sp = jnp.where(first, zero16, sp)
            rv[j] = jnp.where(sws[j], vns[j],
                              jnp.where(sp > 0, vsp, old[j]))
    return rv


def _flat(rk, rv):
    return tuple(rk) + tuple(rv)


def _unflat(c):
    return list(c[:4]), list(c[4:])


def _sc_body(dist_hbm, m_hbm, oidx_hbm, omsk_hbm,
             mrow, drow, candv, candi, obuf_i, obuf_m,
             sem_d0, sem_d1, sem_m0, sem_m1):
    wid = lax.axis_index("s") * 2 + lax.axis_index("c")
    iota = lax.iota(jnp.int32, 16)
    inf16 = jnp.full((16,), jnp.inf, jnp.float32)
    zero16 = jnp.zeros((16,), jnp.int32)
    N = 16384

    def issue(i, buf, sem_d, sem_m):
        @pl.when(i < _QPW)
        def _():
            q = wid * _QPW + i
            pltpu.async_copy(dist_hbm.at[q],
                             drow.at[pl.ds(buf * N, N)], sem_d)
            pltpu.async_copy(m_hbm.at[q],
                             mrow.at[pl.ds(buf * NCH, NCH)], sem_m)

    def process(i, buf, sem_d, sem_m):
        q = wid * _QPW + i
        db = buf * N
        mb = buf * NCH
        pltpu.make_async_copy(
            m_hbm.at[q], mrow.at[pl.ds(mb, NCH)], sem_m).wait()

        rk, rv = [inf16] * 4, [zero16] * 4
        for blk in range(NCH // 64):
            ks = [mrow[pl.ds(mb + blk * 64 + j * 16, 16)] for j in range(4)]
            if blk == 0:
                vs = [iota + (blk * 64 + j * 16) for j in range(4)]
                ak, av = _sort64(ks, vs)
                rk, rv = _merge_keep_low(rk, rv, ak, av)
            else:
                bm = jnp.min(jnp.minimum(jnp.minimum(ks[0], ks[1]),
                                         jnp.minimum(ks[2], ks[3])))
                tcur = jnp.max(rk[3])

                def _merge(args):
                    rk_, rv_, ks_ = list(args[:4]), list(args[4:8]), args[8:]
                    vs = [iota + (blk * 64 + j * 16) for j in range(4)]
                    ak, av = _sort64(list(ks_), vs)
                    return _flat(*_merge_keep_low(rk_, rv_, ak, av))

                def _skip(args):
                    return tuple(args[:8])

                out = lax.cond(bm <= tcur, _merge, _skip,
                               tuple(rk) + tuple(rv) + tuple(ks))
                rk, rv = _unflat(out)
        t = jnp.max(rk[3])

        pltpu.make_async_copy(
            dist_hbm.at[q], drow.at[pl.ds(db, N)], sem_d).wait()

        def cbody(j2, off):
            o = off
            for j in range(4):
                idxv = rv[j] + j2 * NCH
                vals = plsc.load_gather(drow, [idxv + db])
                msk = vals <= t
                cnt = jnp.max(plsc.all_reduce_population_count(msk))
                keyv = jnp.where(vals == 0.0,
                                 idxv.astype(jnp.float32) * jnp.float32(1e-30),
                                 vals)
                plsc.store_compressed(candv.at[pl.ds(o, 16)], keyv, mask=msk)
                plsc.store_compressed(candi.at[pl.ds(o, 16)], idxv, mask=msk)
                o = o + cnt
            return o

        off = lax.fori_loop(0, 32, cbody, jnp.int32(0))

        for j in range(4):
            candv[pl.ds(off + j * 16, 16)] = inf16
            candi[pl.ds(off + j * 16, 16)] = zero16

        nb = (off + 63) // 64

        def fbody(b, carry):
            rk2, rv2 = _unflat(carry)
            ks = [candv[pl.ds(b * 64 + j * 16, 16)] for j in range(4)]
            vs = [candi[pl.ds(b * 64 + j * 16, 16)] for j in range(4)]
            ak, av = _sort64(ks, vs)
            rk2, rv2 = _merge_keep_low(rk2, rv2, ak, av)
            return _flat(rk2, rv2)

        fin = lax.fori_loop(0, nb, fbody, _flat([inf16] * 4, [zero16] * 4))
        rk2, rv2 = _unflat(fin)
        rv2 = _stable_fix(rk2, rv2, iota)

        for j in range(4):
            obuf_i[pl.ds(j * 16, 16)] = rv2[j]
            obuf_m[pl.ds(j * 16, 16)] = jnp.where(
                rk2[j] < jnp.inf, jnp.float32(1.0), jnp.float32(0.0))
        pltpu.sync_copy(obuf_i, oidx_hbm.at[q])
        pltpu.sync_copy(obuf_m, omsk_hbm.at[q])

    issue(jnp.int32(0), 0, sem_d0, sem_m0)
    issue(jnp.int32(1), 1, sem_d1, sem_m1)

    def qbody(i, _carry):
        process(2 * i, 0, sem_d0, sem_m0)
        issue(2 * i + 2, 0, sem_d0, sem_m0)
        process(2 * i + 1, 1, sem_d1, sem_m1)
        issue(2 * i + 3, 1, sem_d1, sem_m1)
        return _carry

    lax.fori_loop(0, _QPW // 2, qbody, jnp.int32(0))


def _sc_select(dist, mins):
    BG = dist.shape[0]
    mesh = plsc.VectorSubcoreMesh(core_axis_name="c", subcore_axis_name="s")
    f = pl.kernel(
        _sc_body,
        mesh=mesh,
        compiler_params=pltpu.CompilerParams(needs_layout_passes=False),
        out_type=[
            jax.ShapeDtypeStruct((BG, K), jnp.int32),
            jax.ShapeDtypeStruct((BG, K), jnp.float32),
        ],
        scratch_types=[
            pltpu.VMEM((2 * NCH,), jnp.float32),
            pltpu.VMEM((2 * 16384,), jnp.float32),
            pltpu.VMEM((_CCAP,), jnp.float32),
            pltpu.VMEM((_CCAP,), jnp.int32),
            pltpu.VMEM((K,), jnp.int32),
            pltpu.VMEM((K,), jnp.float32),
            pltpu.SemaphoreType.DMA,
            pltpu.SemaphoreType.DMA,
            pltpu.SemaphoreType.DMA,
            pltpu.SemaphoreType.DMA,
        ],
    )
    return f(dist, mins)


@jax.jit
def kernel(grasp_translations, scene_xyz, scene_mask):
    q = grasp_translations
    kp = scene_xyz
    B, G, _ = q.shape
    N = kp.shape[1]
    qn = (q * q).sum(-1, keepdims=True)
    kn = (kp * kp).sum(-1)[:, None, :]
    qb = jnp.pad(-2.0 * q, ((0, 0), (0, 0), (0, 5))).astype(jnp.bfloat16)
    kb = jnp.swapaxes(jnp.pad(kp, ((0, 0), (0, 0), (0, 5))), 1, 2).astype(jnp.bfloat16)
    mk = scene_mask[:, None, :]
    dists, mins = _masked_dists(qb, kb, qn, kn, mk)
    idx, msk = _sc_select(dists.reshape(B * G, N), mins.reshape(B * G, NCH))
    return (idx.reshape(B, G, K), msk.reshape(B, G, K))


# speedup vs baseline: 46.3383x; 1.1821x over previous
import functools

import jax
import jax.numpy as jnp
from jax import lax
from jax.experimental import pallas as pl
from jax.experimental.pallas import tpu as pltpu
from jax.experimental.pallas import tpu_sc as plsc

K = 64
NCH = 512
_GB = 256
_NB = 2048
_QPW = 128
_CCAP = 2048 + 80


def _dist_body(qb, kb, qn, kn, mk, dout, mout):
    n = pl.program_id(2)
    s = jnp.dot(qb[0], kb[0], preferred_element_type=jnp.float32)
    d2 = (qn[0] + kn[0]) + s
    dist = jnp.sqrt(jnp.maximum(d2, 0.0))
    dist = jnp.where(mk[0] == 0, jnp.inf, dist)
    dout[0] = dist
    h = jnp.minimum(dist[:, :1024], dist[:, 1024:])
    h = jnp.minimum(h[:, :512], h[:, 512:])

    @pl.when(n == 0)
    def _():
        mout[0] = h

    @pl.when(n != 0)
    def _():
        mout[0] = jnp.minimum(mout[0], h)


def _masked_dists(qb, kb, qn, kn, mk):
    B, G, _ = qb.shape
    N = kb.shape[2]
    grid = (B, G // _GB, N // _NB)
    return pl.pallas_call(
        _dist_body,
        grid=grid,
        in_specs=[
            pl.BlockSpec((1, _GB, 8), lambda b, g, n: (b, g, 0)),
            pl.BlockSpec((1, 8, _NB), lambda b, g, n: (b, 0, n)),
            pl.BlockSpec((1, _GB, 1), lambda b, g, n: (b, g, 0)),
            pl.BlockSpec((1, 1, _NB), lambda b, g, n: (b, 0, n)),
            pl.BlockSpec((1, 1, _NB), lambda b, g, n: (b, 0, n)),
        ],
        out_specs=[
            pl.BlockSpec((1, _GB, _NB), lambda b, g, n: (b, g, n)),
            pl.BlockSpec((1, _GB, NCH), lambda b, g, n: (b, g, 0)),
        ],
        out_shape=[
            jax.ShapeDtypeStruct((B, G, N), jnp.float32),
            jax.ShapeDtypeStruct((B, G, NCH), jnp.float32),
        ],
    )(qb, kb, qn, kn, mk)


def _rev(x):
    return lax.rev(x, (0,))


def _ce(ka, va, kb, vb):
    c = ka <= kb
    return (jnp.where(c, ka, kb), jnp.where(c, va, vb),
            jnp.where(c, kb, ka), jnp.where(c, vb, va))


def _vsort(k, v):
    return plsc.sort_key_val(k, v)


def _sort64(ks, vs):
    ks = [None] * 4 if ks is None else list(ks)
    vs = list(vs)
    for i in range(4):
        ks[i], vs[i] = _vsort(ks[i], vs[i])
    runs = []
    for a, b in ((0, 1), (2, 3)):
        lo_k, lo_v, hi_k, hi_v = _ce(ks[a], vs[a], _rev(ks[b]), _rev(vs[b]))
        lo_k, lo_v = _vsort(lo_k, lo_v)
        hi_k, hi_v = _vsort(hi_k, hi_v)
        runs.append((lo_k, lo_v, hi_k, hi_v))
    (a0k, a0v, a1k, a1v), (b0k, b0v, b1k, b1v) = runs
    l0k, l0v, h0k, h0v = _ce(a0k, a0v, _rev(b1k), _rev(b1v))
    l1k, l1v, h1k, h1v = _ce(a1k, a1v, _rev(b0k), _rev(b0v))
    m0k, m0v, m1k, m1v = _ce(l0k, l0v, l1k, l1v)
    m0k, m0v = _vsort(m0k, m0v)
    m1k, m1v = _vsort(m1k, m1v)
    n0k, n0v, n1k, n1v = _ce(h0k, h0v, h1k, h1v)
    n0k, n0v = _vsort(n0k, n0v)
    n1k, n1v = _vsort(n1k, n1v)
    return [m0k, m1k, n0k, n1k], [m0v, m1v, n0v, n1v]


def _merge_keep_low(rk, rv, ak, av):
    lk, lv = [], []
    for i in range(4):
        k_lo, v_lo, _, _ = _ce(rk[i], rv[i], _rev(ak[3 - i]), _rev(av[3 - i]))
        lk.append(k_lo)
        lv.append(v_lo)
    x0k, x0v, x2k, x2v = _ce(lk[0], lv[0], lk[2], lv[2])
    x1k, x1v, x3k, x3v = _ce(lk[1], lv[1], lk[3], lv[3])
    y0k, y0v, y1k, y1v = _ce(x0k, x0v, x1k, x1v)
    y2k, y2v, y3k, y3v = _ce(x2k, x2v, x3k, x3v)
    ok, ov = [], []
    for kk, vv in ((y0k, y0v), (y1k, y1v), (y2k, y2v), (y3k, y3v)):
        kk, vv = _vsort(kk, vv)
        ok.append(kk)
        ov.append(vv)
    return ok, ov


def _gat(x, idx):
    return x.at[idx].get(mode="promise_in_bounds")


def _stable_fix(rk, rv, iota):
    i32 = jnp.int32
    pu = jnp.minimum(iota + 1, 15)
    pd = jnp.maximum(iota - 1, 0)
    last = iota == 15
    first = iota == 0
    evenm = (iota % 2) == 0
    zero16 = jnp.zeros((16,), i32)
    fifteen = jnp.full((16,), 15, i32)
    for phase in range(4):
        par = evenm if phase % 2 == 0 else jnp.logical_not(evenm)
        old = list(rv)
        sws, vns = [], []
        for j in range(4):
            ks = _gat(rk[j], pu)
            vs = _gat(old[j], pu)
            if j < 3:
                kn = jnp.where(last, _gat(rk[j + 1], zero16), ks)
                vn = jnp.where(last, _gat(old[j + 1], zero16), vs)
            else:
                kn = jnp.where(last, jnp.full((16,), jnp.inf, jnp.float32), ks)
                vn = jnp.where(last, zero16, vs)
            sw = (rk[j] == kn) & (old[j] > vn) & par
            sws.append(sw)
            vns.append(vn)
        for j in range(4):
            swi = sws[j].astype(i32)
            sp = _gat(swi, pd)
            vsp = _gat(old[j], pd)
            if j > 0:
                sp = jnp.where(first, _gat(sws[j - 1].astype(i32), fifteen), sp)
                vsp = jnp.where(first, _gat(old[j - 1], fifteen), vsp)
            else:
                sp = jnp.where(first, zero16, sp)
            rv[j] = jnp.where(sws[j], vns[j],
                              jnp.where(sp > 0, vsp, old[j]))
    return rv


def _flat(rk, rv):
    return tuple(rk) + tuple(rv)


def _unflat(c):
    return list(c[:4]), list(c[4:])


def _sc_body(dist_hbm, m_hbm, oidx_hbm, omsk_hbm,
             mrow, drow, candv, candi, obuf_i, obuf_m,
             sem_d0, sem_d1, sem_m0, sem_m1):
    wid = lax.axis_index("s") * 2 + lax.axis_index("c")
    iota = lax.iota(jnp.int32, 16)
    inf16 = jnp.full((16,), jnp.inf, jnp.float32)
    zero16 = jnp.zeros((16,), jnp.int32)
    N = 16384

    def issue(i, buf, sem_d, sem_m):
        @pl.when(i < _QPW)
        def _():
            q = wid * _QPW + i
            pltpu.async_copy(dist_hbm.at[q],
                             drow.at[pl.ds(buf * N, N)], sem_d)
            pltpu.async_copy(m_hbm.at[q],
                             mrow.at[pl.ds(buf * NCH, NCH)], sem_m)

    def process(i, buf, sem_d, sem_m):
        q = wid * _QPW + i
        db = buf * N
        mb = buf * NCH
        pltpu.make_async_copy(
            m_hbm.at[q], mrow.at[pl.ds(mb, NCH)], sem_m).wait()

        rk, rv = [inf16] * 4, [zero16] * 4
        for blk in range(NCH // 64):
            ks = [mrow[pl.ds(mb + blk * 64 + j * 16, 16)] for j in range(4)]
            if blk == 0:
                vs = [iota + (blk * 64 + j * 16) for j in range(4)]
                ak, av = _sort64(ks, vs)
                rk, rv = _merge_keep_low(rk, rv, ak, av)
            else:
                bm = jnp.min(jnp.minimum(jnp.minimum(ks[0], ks[1]),
                                         jnp.minimum(ks[2], ks[3])))
                tcur = jnp.max(rk[3])

                def _merge(args):
                    rk_, rv_, ks_ = list(args[:4]), list(args[4:8]), args[8:]
                    vs = [iota + (blk * 64 + j * 16) for j in range(4)]
                    ak, av = _sort64(list(ks_), vs)
                    return _flat(*_merge_keep_low(rk_, rv_, ak, av))

                def _skip(args):
                    return tuple(args[:8])

                out = lax.cond(bm <= tcur, _merge, _skip,
                               tuple(rk) + tuple(rv) + tuple(ks))
                rk, rv = _unflat(out)
        t = jnp.max(rk[3])

        pltpu.make_async_copy(
            dist_hbm.at[q], drow.at[pl.ds(db, N)], sem_d).wait()

        def cbody(j2, off):
            idxs, valss, msks, keys = [], [], [], []
            for j in range(4):
                idxv = rv[j] + j2 * NCH
                vals = plsc.load_gather(drow, [idxv + db])
                msk = vals <= t
                keyv = jnp.where(vals == 0.0,
                                 idxv.astype(jnp.float32) * jnp.float32(1e-30),
                                 vals)
                idxs.append(idxv)
                valss.append(vals)
                msks.append(msk)
                keys.append(keyv)
            cs = [plsc.all_reduce_population_count(m) for m in msks]
            packed = (cs[0] + (cs[1] << 8)) + ((cs[2] << 16) + (cs[3] << 24))
            ps = jnp.max(packed)
            o = off
            for j in range(4):
                plsc.store_compressed(candv.at[pl.ds(o, 16)], keys[j],
                                      mask=msks[j])
                plsc.store_compressed(candi.at[pl.ds(o, 16)], idxs[j],
                                      mask=msks[j])
                o = o + ((ps >> (8 * j)) & 0xFF)
            return o

        off = lax.fori_loop(0, 32, cbody, jnp.int32(0))

        for j in range(4):
            candv[pl.ds(off + j * 16, 16)] = inf16
            candi[pl.ds(off + j * 16, 16)] = zero16

        nb = (off + 63) // 64

        def fbody(b, carry):
            rk2, rv2 = _unflat(carry)
            ks = [candv[pl.ds(b * 64 + j * 16, 16)] for j in range(4)]
            vs = [candi[pl.ds(b * 64 + j * 16, 16)] for j in range(4)]
            ak, av = _sort64(ks, vs)
            rk2, rv2 = _merge_keep_low(rk2, rv2, ak, av)
            return _flat(rk2, rv2)

        fin = lax.fori_loop(0, nb, fbody, _flat([inf16] * 4, [zero16] * 4))
        rk2, rv2 = _unflat(fin)
        rv2 = _stable_fix(rk2, rv2, iota)

        for j in range(4):
            obuf_i[pl.ds(j * 16, 16)] = rv2[j]
            obuf_m[pl.ds(j * 16, 16)] = jnp.where(
                rk2[j] < jnp.inf, jnp.float32(1.0), jnp.float32(0.0))
        pltpu.sync_copy(obuf_i, oidx_hbm.at[q])
        pltpu.sync_copy(obuf_m, omsk_hbm.at[q])

    issue(jnp.int32(0), 0, sem_d0, sem_m0)
    issue(jnp.int32(1), 1, sem_d1, sem_m1)

    def qbody(i, _carry):
        process(2 * i, 0, sem_d0, sem_m0)
        issue(2 * i + 2, 0, sem_d0, sem_m0)
        process(2 * i + 1, 1, sem_d1, sem_m1)
        issue(2 * i + 3, 1, sem_d1, sem_m1)
        return _carry

    lax.fori_loop(0, _QPW // 2, qbody, jnp.int32(0))


def _sc_select(dist, mins):
    BG = dist.shape[0]
    mesh = plsc.VectorSubcoreMesh(core_axis_name="c", subcore_axis_name="s")
    f = pl.kernel(
        _sc_body,
        mesh=mesh,
        compiler_params=pltpu.CompilerParams(needs_layout_passes=False),
        out_type=[
            jax.ShapeDtypeStruct((BG, K), jnp.int32),
            jax.ShapeDtypeStruct((BG, K), jnp.float32),
        ],
        scratch_types=[
            pltpu.VMEM((2 * NCH,), jnp.float32),
            pltpu.VMEM((2 * 16384,), jnp.float32),
            pltpu.VMEM((_CCAP,), jnp.float32),
            pltpu.VMEM((_CCAP,), jnp.int32),
            pltpu.VMEM((K,), jnp.int32),
            pltpu.VMEM((K,), jnp.float32),
            pltpu.SemaphoreType.DMA,
            pltpu.SemaphoreType.DMA,
            pltpu.SemaphoreType.DMA,
            pltpu.SemaphoreType.DMA,
        ],
    )
    return f(dist, mins)


@jax.jit
def kernel(grasp_translations, scene_xyz, scene_mask):
    q = grasp_translations
    kp = scene_xyz
    B, G, _ = q.shape
    N = kp.shape[1]
    qn = (q * q).sum(-1, keepdims=True)
    kn = (kp * kp).sum(-1)[:, None, :]
    qb = jnp.pad(-2.0 * q, ((0, 0), (0, 0), (0, 5))).astype(jnp.bfloat16)
    kb = jnp.swapaxes(jnp.pad(kp, ((0, 0), (0, 0), (0, 5))), 1, 2).astype(jnp.bfloat16)
    mk = scene_mask[:, None, :]
    dists, mins = _masked_dists(qb, kb, qn, kn, mk)
    idx, msk = _sc_select(dists.reshape(B * G, N), mins.reshape(B * G, NCH))
    return (idx.reshape(B, G, K), msk.reshape(B, G, K))
